# baseline (speedup 1.0000x reference)
import jax
import jax.numpy as jnp
from jax.experimental import pallas as pl

_DEPTH = 1000
_ROWS = 16384
_BLK = 512


def _onehot_body(x_ref, o_ref):
    idx = x_ref[:, 0]
    iota = jax.lax.broadcasted_iota(jnp.int32, (_BLK, _DEPTH), 1)
    o_ref[:, 0, :] = (iota == idx[:, None]).astype(jnp.float32)


def kernel(x):
    xi = x.astype(jnp.int32)
    out = pl.pallas_call(
        _onehot_body,
        grid=(_ROWS // _BLK,),
        in_specs=[pl.BlockSpec((_BLK, 1), lambda i: (i, 0))],
        out_specs=pl.BlockSpec((_BLK, 1, _DEPTH), lambda i: (i, 0, 0)),
        out_shape=jax.ShapeDtypeStruct((_ROWS, 1, _DEPTH), jnp.float32),
    )(xi)
    return out


# speedup vs baseline: 2.0256x; 2.0256x over previous
import jax
import jax.numpy as jnp
from jax.experimental import pallas as pl

_DEPTH = 1000
_ROWS = 16384
_BLK = 2048


def _onehot_body(x_ref, o_ref):
    idx = x_ref[:, 0]
    iota = jax.lax.broadcasted_iota(jnp.int32, (_BLK, _DEPTH), 1)
    o_ref[...] = (iota == idx[:, None]).astype(jnp.float32)


def kernel(x):
    xi = x.astype(jnp.int32)
    out = pl.pallas_call(
        _onehot_body,
        grid=(_ROWS // _BLK,),
        in_specs=[pl.BlockSpec((_BLK, 1), lambda i: (i, 0))],
        out_specs=pl.BlockSpec((_BLK, _DEPTH), lambda i: (i, 0)),
        out_shape=jax.ShapeDtypeStruct((_ROWS, _DEPTH), jnp.float32),
    )(xi)
    return out.reshape(_ROWS, 1, _DEPTH)


# speedup vs baseline: 8.6195x; 4.2553x over previous
import jax
import jax.numpy as jnp
from jax.experimental import pallas as pl

_DEPTH = 1000
_ROWS = 16384
_RBLK = 2048


def _onehot_body(x_ref, o_ref):
    idx = x_ref[...]
    iota = jax.lax.broadcasted_iota(jnp.int32, (_DEPTH, _RBLK), 0)
    o_ref[...] = (iota == idx).astype(jnp.float32)


def kernel(x):
    xi = x.astype(jnp.int32).reshape(1, _ROWS)
    out = pl.pallas_call(
        _onehot_body,
        grid=(_ROWS // _RBLK,),
        in_specs=[pl.BlockSpec((1, _RBLK), lambda i: (0, i))],
        out_specs=pl.BlockSpec((_DEPTH, _RBLK), lambda i: (0, i)),
        out_shape=jax.ShapeDtypeStruct((_DEPTH, _ROWS), jnp.float32),
    )(xi)
    return out.T.reshape(_ROWS, 1, _DEPTH)
